# baseline (speedup 1.0000x reference)
import functools

import jax
import jax.numpy as jnp
from jax import lax
from jax.experimental import pallas as pl
from jax.experimental.pallas import tpu as pltpu
from jax.experimental.pallas import tpu_sc as plsc

V = 100000
D = 64
B = 1024
CTX = 20
MAXN = 1.0
R = B * CTX

RB = 16384
HP = RB // 2
GRID_R = (V + RB - 1) // RB
VP = GRID_R * HP

NC, NS, LANES = 2, 16, 16
NW = NC * NS
IPW = R // NW
CHUNK = 128
NCHUNK = IPW // CHUNK

BV = 2048
GRID_V = (V + BV - 1) // BV


def _tr_body(tt_ref, o_ref):
    tt = tt_ref[...]
    n2 = jnp.sum(tt * tt, axis=0, keepdims=True)
    s = jnp.minimum(
        jnp.float32(MAXN), lax.rsqrt(jnp.maximum(n2, jnp.float32(1e-24)))
    )
    tt = tt * s
    eye = (
        lax.broadcasted_iota(jnp.int32, (D, D), 0)
        == lax.broadcasted_iota(jnp.int32, (D, D), 1)
    ).astype(jnp.float32)
    left = lax.slice(tt, (0, 0), (D, HP))
    right = lax.slice(tt, (0, HP), (D, RB))

    def tr(m):
        return lax.dot_general(
            m, eye, dimension_numbers=(((0,), (0,)), ((), ())),
            preferred_element_type=jnp.float32,
        )

    o_ref[...] = jnp.concatenate([tr(left), tr(right)], axis=1)


def _repack_rows(tt):
    return pl.pallas_call(
        _tr_body,
        grid=(GRID_R,),
        in_specs=[pl.BlockSpec((D, RB), lambda v: (0, v))],
        out_specs=pl.BlockSpec((HP, 2 * D), lambda v: (v, 0)),
        out_shape=jax.ShapeDtypeStruct((VP, 2 * D), jnp.float32),
    )(tt)


_mesh = plsc.VectorSubcoreMesh(
    core_axis_name="c", subcore_axis_name="s", num_cores=NC, num_subcores=NS
)


@functools.partial(
    pl.kernel,
    out_type=jax.ShapeDtypeStruct((R, 2 * D), jnp.float32),
    mesh=_mesh,
    scratch_types=[
        pltpu.VMEM((NCHUNK, CHUNK), jnp.int32),
        pltpu.VMEM((NCHUNK, CHUNK), jnp.int32),
        pltpu.VMEM((IPW, 2 * D), jnp.float32),
        pltpu.SemaphoreType.DMA,
    ],
    compiler_params=pltpu.CompilerParams(
        needs_layout_passes=False, use_tc_tiling_on_sc=False
    ),
)
def _sc_gather(idx_hbm, table_hbm, rows_hbm, idx_v, pidx_v, rows_v, sem):
    wid = lax.axis_index("s") * NC + lax.axis_index("c")
    pltpu.sync_copy(idx_hbm.at[wid], idx_v)
    for j in range(NCHUNK):
        for k in range(CHUNK // LANES):
            sl = pl.ds(k * LANES, LANES)
            pidx_v[j, sl] = lax.shift_right_logical(idx_v[j, sl], 1)
    copies = [
        pltpu.async_copy(
            table_hbm.at[pidx_v.at[j]], rows_v.at[pl.ds(j * CHUNK, CHUNK)], sem
        )
        for j in range(NCHUNK)
    ]
    for c in copies:
        c.wait()

    for j in range(NCHUNK):
        def canon(k, carry, j=j):
            hb = idx_v[j, pl.ds(k * LANES, LANES)] & 1
            row_ids = j * CHUNK + k * LANES + lax.iota(jnp.int32, LANES)
            colbase = hb * D
            for d in range(D):
                v = plsc.load_gather(rows_v, [row_ids, colbase + d])
                plsc.store_scatter(
                    rows_v, [row_ids, jnp.full((LANES,), d, jnp.int32)], v
                )
            return carry

        lax.fori_loop(0, CHUNK // LANES, canon, 0)

    pltpu.sync_copy(rows_v, rows_hbm.at[pl.ds(wid * IPW, IPW)])


def _mm_body(rows_ref, wt_ref, b_ref, o_ref, x_scr):
    @pl.when(pl.program_id(0) == 0)
    def _pool():
        acc = jnp.zeros((B, D), jnp.float32)
        for c in range(CTX):
            acc = acc + rows_ref[pl.ds(c * B, B), :][:, :D]
        x_scr[...] = acc * jnp.float32(1.0 / CTX)

    o_ref[...] = (
        lax.dot_general(
            wt_ref[...],
            x_scr[...],
            dimension_numbers=(((0,), (1,)), ((), ())),
            preferred_element_type=jnp.float32,
        )
        + jnp.transpose(b_ref[0])
    )


def _project_t(rows, Wt, b_rows):
    return pl.pallas_call(
        _mm_body,
        grid=(GRID_V,),
        in_specs=[
            pl.BlockSpec((R, 2 * D), lambda v: (0, 0)),
            pl.BlockSpec((D, BV), lambda v: (0, v)),
            pl.BlockSpec((1, 1, BV), lambda v: (v, 0, 0)),
        ],
        out_specs=pl.BlockSpec((BV, B), lambda v: (v, 0)),
        out_shape=jax.ShapeDtypeStruct((V, B), jnp.float32),
        scratch_shapes=[pltpu.VMEM((B, D), jnp.float32)],
    )(rows, Wt, b_rows)


def kernel(inputs_, emb_table, W, b):
    jt = inputs_.astype(jnp.int32).T
    enc = ((jt >> 14) << 14) + ((jt & (HP - 1)) << 1) + ((jt >> 13) & 1)
    idx = enc.reshape(NW, NCHUNK, CHUNK)
    table_pack = _repack_rows(emb_table.T)
    rows = _sc_gather(idx, table_pack)
    b_rows = jnp.pad(b, (0, GRID_V * BV - V)).reshape(GRID_V, 1, BV)
    out_t = _project_t(rows, W.T, b_rows)
    return out_t.T


# speedup vs baseline: 1.2801x; 1.2801x over previous
import functools

import jax
import jax.numpy as jnp
from jax import lax
from jax.experimental import pallas as pl
from jax.experimental.pallas import tpu as pltpu
from jax.experimental.pallas import tpu_sc as plsc

V = 100000
D = 64
B = 1024
CTX = 20
MAXN = 1.0
R = B * CTX

RB = 16384
HP = RB // 2
GRID_R = (V + RB - 1) // RB
VP = GRID_R * HP

NC, NS, LANES = 2, 16, 16
NW = NC * NS
IPW = R // NW
CHUNK = 128
NCHUNK = IPW // CHUNK

BV = 2048
GRID_V = (V + BV - 1) // BV


def _tr_body(tt_ref, o_ref):
    tt = tt_ref[...]
    n2 = jnp.sum(tt * tt, axis=0, keepdims=True)
    s = jnp.minimum(
        jnp.float32(MAXN), lax.rsqrt(jnp.maximum(n2, jnp.float32(1e-24)))
    )
    tt = tt * s
    eye = (
        lax.broadcasted_iota(jnp.int32, (D, D), 0)
        == lax.broadcasted_iota(jnp.int32, (D, D), 1)
    ).astype(jnp.float32)
    left = lax.slice(tt, (0, 0), (D, HP))
    right = lax.slice(tt, (0, HP), (D, RB))

    def tr(m):
        return lax.dot_general(
            m, eye, dimension_numbers=(((0,), (0,)), ((), ())),
            preferred_element_type=jnp.float32,
        )

    o_ref[...] = jnp.concatenate([tr(left), tr(right)], axis=1)


def _repack_rows(tt):
    return pl.pallas_call(
        _tr_body,
        grid=(GRID_R,),
        in_specs=[pl.BlockSpec((D, RB), lambda v: (0, v))],
        out_specs=pl.BlockSpec((HP, 2 * D), lambda v: (v, 0)),
        out_shape=jax.ShapeDtypeStruct((VP, 2 * D), jnp.float32),
    )(tt)


_mesh = plsc.VectorSubcoreMesh(
    core_axis_name="c", subcore_axis_name="s", num_cores=NC, num_subcores=NS
)


@functools.partial(
    pl.kernel,
    out_type=jax.ShapeDtypeStruct((B, D), jnp.float32),
    mesh=_mesh,
    scratch_types=[
        pltpu.VMEM((NCHUNK, CHUNK), jnp.int32),
        pltpu.VMEM((IPW, D), jnp.float32),
        pltpu.VMEM((B // NW, D), jnp.float32),
        pltpu.SemaphoreType.DMA,
    ],
    compiler_params=pltpu.CompilerParams(
        needs_layout_passes=False, use_tc_tiling_on_sc=False
    ),
)
def _sc_pool(idx_hbm, table_hbm, x_hbm, idx_v, rows_v, out_v, sem):
    wid = lax.axis_index("s") * NC + lax.axis_index("c")
    bpw = B // NW
    pltpu.sync_copy(idx_hbm.at[wid], idx_v)
    copies = [
        pltpu.async_copy(
            table_hbm.at[idx_v.at[j]], rows_v.at[pl.ds(j * CHUNK, CHUNK)], sem
        )
        for j in range(NCHUNK)
    ]
    for c in copies:
        c.wait()

    def mean_body(bi, carry):
        r0 = bi * CTX
        acc = [jnp.zeros((LANES,), jnp.float32) for _ in range(D // LANES)]
        for c in range(CTX):
            for k in range(D // LANES):
                acc[k] = acc[k] + rows_v[r0 + c, pl.ds(k * LANES, LANES)]
        for k in range(D // LANES):
            out_v[bi, pl.ds(k * LANES, LANES)] = acc[k] * jnp.float32(1.0 / CTX)
        return carry

    lax.fori_loop(0, bpw, mean_body, 0)
    pltpu.sync_copy(out_v, x_hbm.at[pl.ds(wid * bpw, bpw)])


def _mm_body(x_ref, wt_ref, b_ref, o_ref):
    o_ref[...] = (
        lax.dot_general(
            wt_ref[...],
            x_ref[...],
            dimension_numbers=(((0,), (1,)), ((), ())),
            preferred_element_type=jnp.float32,
        )
        + jnp.transpose(b_ref[0])
    )


def _project_t(x, Wt, b_rows):
    return pl.pallas_call(
        _mm_body,
        grid=(GRID_V,),
        in_specs=[
            pl.BlockSpec((B, D), lambda v: (0, 0)),
            pl.BlockSpec((D, BV), lambda v: (0, v)),
            pl.BlockSpec((1, 1, BV), lambda v: (v, 0, 0)),
        ],
        out_specs=pl.BlockSpec((BV, B), lambda v: (v, 0)),
        out_shape=jax.ShapeDtypeStruct((V, B), jnp.float32),
    )(x, Wt, b_rows)


def kernel(inputs_, emb_table, W, b):
    jt = inputs_.astype(jnp.int32)
    enc = ((jt >> 14) << 14) + ((jt & (HP - 1)) << 1) + ((jt >> 13) & 1)
    idx = enc.reshape(NW, NCHUNK, CHUNK)
    table64 = _repack_rows(emb_table.T).reshape(2 * VP, D)
    x = _sc_pool(idx, table64)
    b_rows = jnp.pad(b, (0, GRID_V * BV - V)).reshape(GRID_V, 1, BV)
    out_t = _project_t(x, W.T, b_rows)
    return out_t.T
